# baseline (speedup 1.0000x reference)
import functools

import jax
import jax.numpy as jnp
from jax import lax
from jax.experimental import pallas as pl
from jax.experimental.pallas import tpu as pltpu
from jax.experimental.pallas import tpu_sc as plsc

NC = 2
NS = 16
NW = NC * NS


@functools.lru_cache(maxsize=None)
def _make_deg(N, E, CH, K):
    RPT = N // NS
    mesh = plsc.VectorSubcoreMesh(core_axis_name="c", subcore_axis_name="s")

    FIRE = 5
    assert CH % FIRE == 0

    @functools.partial(
        pl.kernel,
        out_type=jax.ShapeDtypeStruct((NC, N, 128), jnp.float32),
        mesh=mesh,
        scratch_types=[
            pltpu.VMEM((CH, 128), jnp.int32),
            pltpu.VMEM((128, 128), jnp.float32),
            pltpu.VMEM_SHARED((N, 128), jnp.float32),
            pltpu.SemaphoreType.DMA,
        ],
    )
    def deg_kernel(rows_hbm, ones_hbm, zeros_hbm, out_hbm, ridx, ones_v, acc,
                   ssem):
        c = lax.axis_index("c")
        s = lax.axis_index("s")
        w = c * NS + s
        pltpu.sync_copy(rows_hbm.at[w], ridx)
        pltpu.sync_copy(ones_hbm, ones_v)
        pltpu.sync_copy(zeros_hbm, acc.at[pl.ds(s * RPT, RPT)])
        plsc.subcore_barrier()

        def body(jj, carry):
            for t in range(FIRE):
                pltpu.async_copy(ones_v, acc.at[ridx.at[jj * FIRE + t]], ssem,
                                 add=True)
            for t in range(FIRE):
                pltpu.make_async_copy(ones_v, acc.at[ridx.at[0]], ssem).wait()
            return carry

        lax.fori_loop(0, CH // FIRE, body, 0)
        plsc.subcore_barrier()
        pltpu.sync_copy(acc.at[pl.ds(s * RPT, RPT)],
                        out_hbm.at[c, pl.ds(s * RPT, RPT)])

    return deg_kernel


@functools.lru_cache(maxsize=None)
def _make_prop(N, E, CH):
    RPT = N // NS
    mesh = plsc.VectorSubcoreMesh(core_axis_name="c", subcore_axis_name="s")

    @functools.partial(
        pl.kernel,
        out_type=jax.ShapeDtypeStruct((NC, N, 128), jnp.float32),
        mesh=mesh,
        scratch_types=[
            pltpu.VMEM((CH, 128), jnp.int32),
            pltpu.VMEM((CH, 128), jnp.int32),
            pltpu.VMEM((128, 128), jnp.float32),
            pltpu.SemaphoreType.DMA,
            pltpu.VMEM_SHARED((N, 128), jnp.float32),
        ],
    )
    def prop_kernel(rows_hbm, cols_hbm, table_hbm, zeros_hbm, out_hbm,
                    ridx, cidx, buf, gsem, acc):
        c = lax.axis_index("c")
        s = lax.axis_index("s")
        w = c * NS + s
        pltpu.sync_copy(rows_hbm.at[w], ridx)
        pltpu.sync_copy(cols_hbm.at[w], cidx)
        pltpu.sync_copy(zeros_hbm, acc.at[pl.ds(s * RPT, RPT)])
        plsc.subcore_barrier()

        def body(j, carry):
            pltpu.async_copy(table_hbm.at[ridx.at[j]], buf, gsem).wait()
            pltpu.sync_copy(buf, acc.at[cidx.at[j]], add=True)
            return carry

        lax.fori_loop(0, CH, body, 0)
        plsc.subcore_barrier()
        pltpu.sync_copy(acc.at[pl.ds(s * RPT, RPT)],
                        out_hbm.at[c, pl.ds(s * RPT, RPT)])

    return prop_kernel


def _tc_prep(d0, d1, x, R):
    N, F = x.shape

    def body(d0_ref, d1_ref, x_ref, dinv_ref, g0_ref):
        deg = d0_ref[:, 0:1] + d1_ref[:, 0:1]
        dinv = jnp.where(deg > 0.0, lax.rsqrt(jnp.maximum(deg, 1e-30)), 0.0)
        dinv_ref[...] = dinv
        g0_ref[...] = dinv * x_ref[...]

    grid = N // R
    return pl.pallas_call(
        body,
        grid=(grid,),
        in_specs=[
            pl.BlockSpec((R, d0.shape[1]), lambda i: (i, 0)),
            pl.BlockSpec((R, d1.shape[1]), lambda i: (i, 0)),
            pl.BlockSpec((R, F), lambda i: (i, 0)),
        ],
        out_specs=[
            pl.BlockSpec((R, 1), lambda i: (i, 0)),
            pl.BlockSpec((R, F), lambda i: (i, 0)),
        ],
        out_shape=[
            jax.ShapeDtypeStruct((N, 1), jnp.float32),
            jax.ShapeDtypeStruct((N, F), jnp.float32),
        ],
    )(d0, d1, x)


def _tc_mid(p0, p1, dinv, R):
    N, F = p0.shape

    def body(p0_ref, p1_ref, dinv_ref, tx1_ref, g1_ref):
        dv = dinv_ref[...]
        tx1 = -dv * (p0_ref[...] + p1_ref[...])
        tx1_ref[...] = tx1
        g1_ref[...] = dv * tx1

    grid = N // R
    return pl.pallas_call(
        body,
        grid=(grid,),
        in_specs=[
            pl.BlockSpec((R, F), lambda i: (i, 0)),
            pl.BlockSpec((R, F), lambda i: (i, 0)),
            pl.BlockSpec((R, 1), lambda i: (i, 0)),
        ],
        out_specs=[
            pl.BlockSpec((R, F), lambda i: (i, 0)),
            pl.BlockSpec((R, F), lambda i: (i, 0)),
        ],
        out_shape=[
            jax.ShapeDtypeStruct((N, F), jnp.float32),
            jax.ShapeDtypeStruct((N, F), jnp.float32),
        ],
    )(p0, p1, dinv)


def _tc_layer(q0, q1, dinv, tx0, tx1, Wc, b, R, emit_next):
    N, F = tx0.shape
    H = Wc.shape[2]

    def body(q0_ref, q1_ref, dinv_ref, tx0_ref, tx1_ref, w_ref, b_ref, *outs):
        dv = dinv_ref[...]
        tx0v = tx0_ref[...]
        tx2 = -2.0 * dv * (q0_ref[...] + q1_ref[...]) - tx0v
        out = (jnp.dot(tx0v, w_ref[0], preferred_element_type=jnp.float32)
               + jnp.dot(tx1_ref[...], w_ref[1], preferred_element_type=jnp.float32)
               + jnp.dot(tx2, w_ref[2], preferred_element_type=jnp.float32)
               + b_ref[...])
        outs[0][...] = out
        if emit_next:
            hn = jnp.maximum(out, 0.0)
            outs[1][...] = hn
            outs[2][...] = dv * hn

    grid = N // R
    n_out = 3 if emit_next else 1
    return pl.pallas_call(
        body,
        grid=(grid,),
        in_specs=[
            pl.BlockSpec((R, F), lambda i: (i, 0)),
            pl.BlockSpec((R, F), lambda i: (i, 0)),
            pl.BlockSpec((R, 1), lambda i: (i, 0)),
            pl.BlockSpec((R, F), lambda i: (i, 0)),
            pl.BlockSpec((R, F), lambda i: (i, 0)),
            pl.BlockSpec((3, F, H), lambda i: (0, 0, 0)),
            pl.BlockSpec((1, H), lambda i: (0, 0)),
        ],
        out_specs=[pl.BlockSpec((R, H), lambda i: (i, 0))] * n_out,
        out_shape=[jax.ShapeDtypeStruct((N, H), jnp.float32)] * n_out,
    )(q0, q1, dinv, tx0, tx1, Wc, b)


def _tc_pool(h, batch_f, lin_w, lin_b, C):
    N, H = h.shape
    G = 64
    OUT = lin_w.shape[0]
    grid = N // C

    def body(h_ref, b_ref, w_ref, lb_ref, pooled_ref, out_ref, sums, cnt):
        i = pl.program_id(0)
        gids = lax.broadcasted_iota(jnp.int32, (G, C), 0).astype(jnp.float32)
        oh = jnp.where(gids == b_ref[0], 1.0, 0.0)
        psum = jnp.dot(oh, h_ref[...], preferred_element_type=jnp.float32)
        pcnt = jnp.sum(oh, axis=1, keepdims=True)

        @pl.when(i == 0)
        def _():
            sums[...] = psum
            cnt[...] = pcnt

        @pl.when(i > 0)
        def _():
            sums[...] = sums[...] + psum
            cnt[...] = cnt[...] + pcnt

        @pl.when(i == grid - 1)
        def _():
            pooled = sums[...] / jnp.maximum(cnt[...], 1.0)
            pooled_ref[...] = pooled
            out_ref[...] = lax.dot_general(
                pooled, w_ref[...], (((1,), (1,)), ((), ())),
                preferred_element_type=jnp.float32) + lb_ref[...]

    return pl.pallas_call(
        body,
        grid=(grid,),
        in_specs=[
            pl.BlockSpec((C, H), lambda i: (i, 0)),
            pl.BlockSpec((1, 1, C), lambda i: (i, 0, 0)),
            pl.BlockSpec((OUT, H), lambda i: (0, 0)),
            pl.BlockSpec((1, OUT), lambda i: (0, 0)),
        ],
        out_specs=[
            pl.BlockSpec((G, H), lambda i: (0, 0)),
            pl.BlockSpec((G, OUT), lambda i: (0, 0)),
        ],
        out_shape=[
            jax.ShapeDtypeStruct((G, H), jnp.float32),
            jax.ShapeDtypeStruct((G, OUT), jnp.float32),
        ],
        scratch_shapes=[
            pltpu.VMEM((G, H), jnp.float32),
            pltpu.VMEM((G, 1), jnp.float32),
        ],
    )(h, batch_f, lin_w, lin_b)


def kernel(x, edge_index, batch, W1, b1, W2, b2, W3, b3, lin_w, lin_b):
    N, D = x.shape
    E = edge_index.shape[1]
    H = W1.shape[1]
    K = 125
    CH = E // (NW * K)
    R = 2000
    NP = ((N + 127) // 128 + 1) * 128
    RPT = NP // NS

    rows3 = edge_index[0].reshape(NW, CH, K)
    cols3 = edge_index[1].reshape(NW, CH, K)
    JL = 128 - K
    pad3 = (N + jnp.arange(NW * CH * JL, dtype=jnp.int32) % (NP - N)
            ).reshape(NW, CH, JL)
    rows3p = jnp.pad(rows3, ((0, 0), (0, 0), (0, JL)))
    cols3p = jnp.concatenate([cols3, pad3], axis=2)
    rowsdeg = jnp.concatenate([rows3, pad3], axis=2)
    zeros_h = jnp.zeros((RPT, H), jnp.float32)
    ones_h = jnp.ones((128, 128), jnp.float32)

    deg_k = _make_deg(NP, E, CH, K)
    prop_k = _make_prop(NP, E, CH)

    dpart = deg_k(rowsdeg, ones_h, zeros_h)
    dinv, g = _tc_prep(dpart[0, :N], dpart[1, :N], x, R)

    xs = []
    h = x
    for li, (W, b) in enumerate(((W1, b1), (W2, b2), (W3, b3))):
        Wc = jnp.transpose(W, (0, 2, 1))
        p = prop_k(rows3p, cols3p, g, zeros_h)
        tx1, g1 = _tc_mid(p[0, :N], p[1, :N], dinv, R)
        q = prop_k(rows3p, cols3p, g1, zeros_h)
        last = li == 2
        outs = _tc_layer(q[0, :N], q[1, :N], dinv, h, tx1, Wc,
                         b.reshape(1, H), R, emit_next=not last)
        xs.append(outs[0])
        if not last:
            h, g = outs[1], outs[2]

    h3 = xs[2]
    pooled, out = _tc_pool(h3, batch.astype(jnp.float32).reshape(N // 2000, 1, 2000),
                           lin_w, lin_b.reshape(1, lin_w.shape[0]), 2000)
    return (out, xs[0], xs[1], h3, pooled)


# speedup vs baseline: 1.9009x; 1.9009x over previous
import functools

import jax
import jax.numpy as jnp
from jax import lax
from jax.experimental import pallas as pl
from jax.experimental.pallas import tpu as pltpu
from jax.experimental.pallas import tpu_sc as plsc

NC = 2
NS = 16
NW = NC * NS


@functools.lru_cache(maxsize=None)
def _make_deg(N, E, CH, K):
    RPT = N // NS
    mesh = plsc.VectorSubcoreMesh(core_axis_name="c", subcore_axis_name="s")

    FIRE = 5
    assert CH % FIRE == 0

    @functools.partial(
        pl.kernel,
        out_type=jax.ShapeDtypeStruct((NC, N, 128), jnp.float32),
        mesh=mesh,
        scratch_types=[
            pltpu.VMEM((CH, 128), jnp.int32),
            pltpu.VMEM((128, 128), jnp.float32),
            pltpu.VMEM_SHARED((N, 128), jnp.float32),
            pltpu.SemaphoreType.DMA,
        ],
    )
    def deg_kernel(rows_hbm, ones_hbm, zeros_hbm, out_hbm, ridx, ones_v, acc,
                   ssem):
        c = lax.axis_index("c")
        s = lax.axis_index("s")
        w = c * NS + s
        pltpu.sync_copy(rows_hbm.at[w], ridx)
        pltpu.sync_copy(ones_hbm, ones_v)
        pltpu.sync_copy(zeros_hbm, acc.at[pl.ds(s * RPT, RPT)])
        plsc.subcore_barrier()

        def body(jj, carry):
            for t in range(FIRE):
                pltpu.async_copy(ones_v, acc.at[ridx.at[jj * FIRE + t]], ssem,
                                 add=True)
            for t in range(FIRE):
                pltpu.make_async_copy(ones_v, acc.at[ridx.at[0]], ssem).wait()
            return carry

        lax.fori_loop(0, CH // FIRE, body, 0)
        plsc.subcore_barrier()
        pltpu.sync_copy(acc.at[pl.ds(s * RPT, RPT)],
                        out_hbm.at[c, pl.ds(s * RPT, RPT)])

    return deg_kernel


@functools.lru_cache(maxsize=None)
def _make_prop(N, E, CH):
    RPT = N // NS
    mesh = plsc.VectorSubcoreMesh(core_axis_name="c", subcore_axis_name="s")

    @functools.partial(
        pl.kernel,
        out_type=jax.ShapeDtypeStruct((NC, N, 128), jnp.float32),
        mesh=mesh,
        scratch_types=[
            pltpu.VMEM((CH, 128), jnp.int32),
            pltpu.VMEM((CH, 128), jnp.int32),
            pltpu.VMEM((128, 128), jnp.float32),
            pltpu.SemaphoreType.DMA,
            pltpu.VMEM_SHARED((N, 128), jnp.float32),
        ],
    )
    def prop_kernel(rows_hbm, cols_hbm, table_hbm, zeros_hbm, out_hbm,
                    ridx, cidx, buf, gsem, acc):
        c = lax.axis_index("c")
        s = lax.axis_index("s")
        w = c * NS + s
        pltpu.sync_copy(rows_hbm.at[w], ridx)
        pltpu.sync_copy(cols_hbm.at[w], cidx)
        pltpu.sync_copy(zeros_hbm, acc.at[pl.ds(s * RPT, RPT)])
        plsc.subcore_barrier()

        def body(j, carry):
            pltpu.async_copy(table_hbm.at[ridx.at[j]], buf, gsem).wait()
            pltpu.sync_copy(buf, acc.at[cidx.at[j]], add=True)
            return carry

        lax.fori_loop(0, CH, body, 0)
        plsc.subcore_barrier()
        pltpu.sync_copy(acc.at[pl.ds(s * RPT, RPT)],
                        out_hbm.at[c, pl.ds(s * RPT, RPT)])

    return prop_kernel


def _tc_prep(d0, d1, x, R):
    N, F = x.shape

    def body(d0_ref, d1_ref, x_ref, dinv_ref, g0_ref):
        deg = d0_ref[:, 0:1] + d1_ref[:, 0:1]
        dinv = jnp.where(deg > 0.0, lax.rsqrt(jnp.maximum(deg, 1e-30)), 0.0)
        dinv_ref[...] = dinv
        g0_ref[...] = dinv * x_ref[...]

    grid = N // R
    return pl.pallas_call(
        body,
        grid=(grid,),
        in_specs=[
            pl.BlockSpec((R, d0.shape[1]), lambda i: (i, 0)),
            pl.BlockSpec((R, d1.shape[1]), lambda i: (i, 0)),
            pl.BlockSpec((R, F), lambda i: (i, 0)),
        ],
        out_specs=[
            pl.BlockSpec((R, 1), lambda i: (i, 0)),
            pl.BlockSpec((R, F), lambda i: (i, 0)),
        ],
        out_shape=[
            jax.ShapeDtypeStruct((N, 1), jnp.float32),
            jax.ShapeDtypeStruct((N, F), jnp.float32),
        ],
    )(d0, d1, x)


def _tc_mid(p0, p1, dinv, R):
    N, F = p0.shape

    def body(p0_ref, p1_ref, dinv_ref, tx1_ref, g1_ref):
        dv = dinv_ref[...]
        tx1 = -dv * (p0_ref[...] + p1_ref[...])
        tx1_ref[...] = tx1
        g1_ref[...] = dv * tx1

    grid = N // R
    return pl.pallas_call(
        body,
        grid=(grid,),
        in_specs=[
            pl.BlockSpec((R, F), lambda i: (i, 0)),
            pl.BlockSpec((R, F), lambda i: (i, 0)),
            pl.BlockSpec((R, 1), lambda i: (i, 0)),
        ],
        out_specs=[
            pl.BlockSpec((R, F), lambda i: (i, 0)),
            pl.BlockSpec((R, F), lambda i: (i, 0)),
        ],
        out_shape=[
            jax.ShapeDtypeStruct((N, F), jnp.float32),
            jax.ShapeDtypeStruct((N, F), jnp.float32),
        ],
    )(p0, p1, dinv)


def _tc_layer(q0, q1, dinv, tx0, tx1, Wc, b, R, emit_next):
    N, F = tx0.shape
    H = Wc.shape[2]

    def body(q0_ref, q1_ref, dinv_ref, tx0_ref, tx1_ref, w_ref, b_ref, *outs):
        dv = dinv_ref[...]
        tx0v = tx0_ref[...]
        tx2 = -2.0 * dv * (q0_ref[...] + q1_ref[...]) - tx0v
        out = (jnp.dot(tx0v, w_ref[0], preferred_element_type=jnp.float32)
               + jnp.dot(tx1_ref[...], w_ref[1], preferred_element_type=jnp.float32)
               + jnp.dot(tx2, w_ref[2], preferred_element_type=jnp.float32)
               + b_ref[...])
        outs[0][...] = out
        if emit_next:
            hn = jnp.maximum(out, 0.0)
            outs[1][...] = hn
            outs[2][...] = dv * hn

    grid = N // R
    n_out = 3 if emit_next else 1
    return pl.pallas_call(
        body,
        grid=(grid,),
        in_specs=[
            pl.BlockSpec((R, F), lambda i: (i, 0)),
            pl.BlockSpec((R, F), lambda i: (i, 0)),
            pl.BlockSpec((R, 1), lambda i: (i, 0)),
            pl.BlockSpec((R, F), lambda i: (i, 0)),
            pl.BlockSpec((R, F), lambda i: (i, 0)),
            pl.BlockSpec((3, F, H), lambda i: (0, 0, 0)),
            pl.BlockSpec((1, H), lambda i: (0, 0)),
        ],
        out_specs=[pl.BlockSpec((R, H), lambda i: (i, 0))] * n_out,
        out_shape=[jax.ShapeDtypeStruct((N, H), jnp.float32)] * n_out,
    )(q0, q1, dinv, tx0, tx1, Wc, b)


def _tc_pool(h, batch_f, lin_w, lin_b, C):
    N, H = h.shape
    G = 64
    OUT = lin_w.shape[0]
    grid = N // C

    def body(h_ref, b_ref, w_ref, lb_ref, pooled_ref, out_ref, sums, cnt):
        i = pl.program_id(0)
        gids = lax.broadcasted_iota(jnp.int32, (G, C), 0).astype(jnp.float32)
        oh = jnp.where(gids == b_ref[0], 1.0, 0.0)
        psum = jnp.dot(oh, h_ref[...], preferred_element_type=jnp.float32)
        pcnt = jnp.sum(oh, axis=1, keepdims=True)

        @pl.when(i == 0)
        def _():
            sums[...] = psum
            cnt[...] = pcnt

        @pl.when(i > 0)
        def _():
            sums[...] = sums[...] + psum
            cnt[...] = cnt[...] + pcnt

        @pl.when(i == grid - 1)
        def _():
            pooled = sums[...] / jnp.maximum(cnt[...], 1.0)
            pooled_ref[...] = pooled
            out_ref[...] = lax.dot_general(
                pooled, w_ref[...], (((1,), (1,)), ((), ())),
                preferred_element_type=jnp.float32) + lb_ref[...]

    return pl.pallas_call(
        body,
        grid=(grid,),
        in_specs=[
            pl.BlockSpec((C, H), lambda i: (i, 0)),
            pl.BlockSpec((1, 1, C), lambda i: (i, 0, 0)),
            pl.BlockSpec((OUT, H), lambda i: (0, 0)),
            pl.BlockSpec((1, OUT), lambda i: (0, 0)),
        ],
        out_specs=[
            pl.BlockSpec((G, H), lambda i: (0, 0)),
            pl.BlockSpec((G, OUT), lambda i: (0, 0)),
        ],
        out_shape=[
            jax.ShapeDtypeStruct((G, H), jnp.float32),
            jax.ShapeDtypeStruct((G, OUT), jnp.float32),
        ],
        scratch_shapes=[
            pltpu.VMEM((G, H), jnp.float32),
            pltpu.VMEM((G, 1), jnp.float32),
        ],
    )(h, batch_f, lin_w, lin_b)


def kernel(x, edge_index, batch, W1, b1, W2, b2, W3, b3, lin_w, lin_b):
    N, D = x.shape
    E = edge_index.shape[1]
    H = W1.shape[1]
    K = 125
    CH = E // (NW * K)
    R = 2000
    NP = ((N + 127) // 128 + 1) * 128
    RPT = NP // NS

    rows3 = edge_index[0].reshape(NW, CH, K)
    cols3 = edge_index[1].reshape(NW, CH, K)
    JL = 128 - K
    spread = jnp.arange(NW * CH * JL, dtype=jnp.int32)
    pad3 = (N + spread % (NP - N)).reshape(NW, CH, JL)
    padg = (spread % N).reshape(NW, CH, JL)
    rows3p = jnp.concatenate([rows3, padg], axis=2)
    cols3p = jnp.concatenate([cols3, pad3], axis=2)
    rowsdeg = jnp.concatenate([rows3, pad3], axis=2)
    zeros_h = jnp.zeros((RPT, H), jnp.float32)
    ones_h = jnp.ones((128, 128), jnp.float32)

    deg_k = _make_deg(NP, E, CH, K)
    prop_k = _make_prop(NP, E, CH)

    dpart = deg_k(rowsdeg, ones_h, zeros_h)
    dinv, g = _tc_prep(dpart[0, :N], dpart[1, :N], x, R)

    xs = []
    h = x
    for li, (W, b) in enumerate(((W1, b1), (W2, b2), (W3, b3))):
        Wc = jnp.transpose(W, (0, 2, 1))
        p = prop_k(rows3p, cols3p, g, zeros_h)
        tx1, g1 = _tc_mid(p[0, :N], p[1, :N], dinv, R)
        q = prop_k(rows3p, cols3p, g1, zeros_h)
        last = li == 2
        outs = _tc_layer(q[0, :N], q[1, :N], dinv, h, tx1, Wc,
                         b.reshape(1, H), R, emit_next=not last)
        xs.append(outs[0])
        if not last:
            h, g = outs[1], outs[2]

    h3 = xs[2]
    pooled, out = _tc_pool(h3, batch.astype(jnp.float32).reshape(N // 2000, 1, 2000),
                           lin_w, lin_b.reshape(1, lin_w.shape[0]), 2000)
    return (out, xs[0], xs[1], h3, pooled)


# speedup vs baseline: 2.4062x; 1.2658x over previous
import functools

import jax
import jax.numpy as jnp
from jax import lax
from jax.experimental import pallas as pl
from jax.experimental.pallas import tpu as pltpu
from jax.experimental.pallas import tpu_sc as plsc

NC = 2
NS = 16
NW = NC * NS


@functools.lru_cache(maxsize=None)
def _make_deg(N, E, CH, K):
    RPT = N // NS
    mesh = plsc.VectorSubcoreMesh(core_axis_name="c", subcore_axis_name="s")

    FIRE = 5
    assert CH % FIRE == 0

    @functools.partial(
        pl.kernel,
        out_type=jax.ShapeDtypeStruct((NC, N, 128), jnp.float32),
        mesh=mesh,
        scratch_types=[
            pltpu.VMEM((CH, 128), jnp.int32),
            pltpu.VMEM((128, 128), jnp.float32),
            pltpu.VMEM_SHARED((N, 128), jnp.float32),
            pltpu.SemaphoreType.DMA,
        ],
    )
    def deg_kernel(rows_hbm, ones_hbm, zeros_hbm, out_hbm, ridx, ones_v, acc,
                   ssem):
        c = lax.axis_index("c")
        s = lax.axis_index("s")
        w = c * NS + s
        pltpu.sync_copy(rows_hbm.at[w], ridx)
        pltpu.sync_copy(ones_hbm, ones_v)
        pltpu.sync_copy(zeros_hbm, acc.at[pl.ds(s * RPT, RPT)])
        plsc.subcore_barrier()

        def body(jj, carry):
            for t in range(FIRE):
                pltpu.async_copy(ones_v, acc.at[ridx.at[jj * FIRE + t]], ssem,
                                 add=True)
            for t in range(FIRE):
                pltpu.make_async_copy(ones_v, acc.at[ridx.at[0]], ssem).wait()
            return carry

        lax.fori_loop(0, CH // FIRE, body, 0)
        plsc.subcore_barrier()
        pltpu.sync_copy(acc.at[pl.ds(s * RPT, RPT)],
                        out_hbm.at[c, pl.ds(s * RPT, RPT)])

    return deg_kernel


@functools.lru_cache(maxsize=None)
def _make_prop(N, E, CH):
    RPT = N // NS
    NQ = CH // 4
    assert CH % 8 == 0 and CH >= 16
    mesh = plsc.VectorSubcoreMesh(core_axis_name="c", subcore_axis_name="s")

    @functools.partial(
        pl.kernel,
        out_type=jax.ShapeDtypeStruct((NC, N, 128), jnp.float32),
        mesh=mesh,
        scratch_types=[
            pltpu.VMEM((8, 128), jnp.int32),
            pltpu.VMEM((8, 128), jnp.int32),
            pltpu.VMEM((128, 128), jnp.float32),
            pltpu.VMEM((128, 128), jnp.float32),
            pltpu.SemaphoreType.DMA,
            pltpu.SemaphoreType.DMA,
            pltpu.SemaphoreType.DMA,
            pltpu.SemaphoreType.DMA,
            pltpu.SemaphoreType.DMA,
            pltpu.VMEM_SHARED((N, 128), jnp.float32),
        ],
    )
    def prop_kernel(rows_hbm, cols_hbm, table_hbm, zeros_hbm, out_hbm,
                    rring, cring, buf0, buf1, gs0, gs1, ss0, ss1, isem, acc):
        c = lax.axis_index("c")
        s = lax.axis_index("s")
        w = c * NS + s
        pltpu.sync_copy(zeros_hbm, acc.at[pl.ds(s * RPT, RPT)])
        pltpu.sync_copy(rows_hbm.at[w, pl.ds(0, 4)], rring.at[pl.ds(0, 4)])
        pltpu.sync_copy(cols_hbm.at[w, pl.ds(0, 4)], cring.at[pl.ds(0, 4)])
        plsc.subcore_barrier()

        bufs = (buf0, buf1)
        gsems = (gs0, gs1)
        ssems = (ss0, ss1)

        def gather(r, b):
            pltpu.async_copy(table_hbm.at[rring.at[r]], bufs[b], gsems[b])

        def scatter(r, b):
            pltpu.async_copy(bufs[b], acc.at[cring.at[r]], ssems[b], add=True)

        def wait_g(b):
            pltpu.make_async_copy(table_hbm.at[rring.at[0]], bufs[b],
                                  gsems[b]).wait()

        def wait_s(b):
            pltpu.make_async_copy(bufs[b], acc.at[cring.at[0]],
                                  ssems[b]).wait()

        def slab_issue(start, h):
            pltpu.async_copy(rows_hbm.at[w, pl.ds(start, 4)],
                             rring.at[pl.ds(4 * h, 4)], isem)
            pltpu.async_copy(cols_hbm.at[w, pl.ds(start, 4)],
                             cring.at[pl.ds(4 * h, 4)], isem)

        def slab_wait():
            for _ in range(2):
                pltpu.make_async_copy(rows_hbm.at[w, pl.ds(0, 4)],
                                      rring.at[pl.ds(0, 4)], isem).wait()

        slab_issue(4, 1)
        gather(0, 0)
        wait_g(0)
        gather(1, 1)
        scatter(0, 0)
        wait_g(1); wait_s(0); gather(2, 0); scatter(1, 1)
        wait_g(0); wait_s(1); gather(3, 1); scatter(2, 0)
        wait_g(1); wait_s(0); slab_wait(); gather(4, 0)
        slab_issue(8, 0)
        scatter(3, 1)

        def quad(q, h):
            wait_g(0); wait_s(1); gather(4 * h + 1, 1); scatter(4 * h + 0, 0)
            wait_g(1); wait_s(0); gather(4 * h + 2, 0); scatter(4 * h + 1, 1)
            wait_g(0); wait_s(1); gather(4 * h + 3, 1); scatter(4 * h + 2, 0)
            wait_g(1); wait_s(0); slab_wait()
            gather(4 * (1 - h), 0)
            slab_issue(jnp.minimum(4 * q + 8, CH - 4), h)
            scatter(4 * h + 3, 1)

        def pair(qq, carry):
            quad(2 * qq + 1, 1)
            quad(2 * qq + 2, 0)
            return carry

        lax.fori_loop(0, (NQ - 2) // 2, pair, 0)

        wait_g(0); wait_s(1); gather(5, 1); scatter(4, 0)
        wait_g(1); wait_s(0); gather(6, 0); scatter(5, 1)
        wait_g(0); wait_s(1); gather(7, 1); scatter(6, 0)
        wait_g(1); wait_s(0); slab_wait()
        scatter(7, 1)
        wait_s(1)

        plsc.subcore_barrier()
        pltpu.sync_copy(acc.at[pl.ds(s * RPT, RPT)],
                        out_hbm.at[c, pl.ds(s * RPT, RPT)])

    return prop_kernel


def _tc_prep(d0, d1, x, R):
    N, F = x.shape

    def body(d0_ref, d1_ref, x_ref, dinv_ref, g0_ref):
        deg = d0_ref[:, 0:1] + d1_ref[:, 0:1]
        dinv = jnp.where(deg > 0.0, lax.rsqrt(jnp.maximum(deg, 1e-30)), 0.0)
        dinv_ref[...] = dinv
        g0_ref[...] = dinv * x_ref[...]

    grid = N // R
    return pl.pallas_call(
        body,
        grid=(grid,),
        in_specs=[
            pl.BlockSpec((R, d0.shape[1]), lambda i: (i, 0)),
            pl.BlockSpec((R, d1.shape[1]), lambda i: (i, 0)),
            pl.BlockSpec((R, F), lambda i: (i, 0)),
        ],
        out_specs=[
            pl.BlockSpec((R, 1), lambda i: (i, 0)),
            pl.BlockSpec((R, F), lambda i: (i, 0)),
        ],
        out_shape=[
            jax.ShapeDtypeStruct((N, 1), jnp.float32),
            jax.ShapeDtypeStruct((N, F), jnp.float32),
        ],
    )(d0, d1, x)


def _tc_mid(p0, p1, dinv, R):
    N, F = p0.shape

    def body(p0_ref, p1_ref, dinv_ref, tx1_ref, g1_ref):
        dv = dinv_ref[...]
        tx1 = -dv * (p0_ref[...] + p1_ref[...])
        tx1_ref[...] = tx1
        g1_ref[...] = dv * tx1

    grid = N // R
    return pl.pallas_call(
        body,
        grid=(grid,),
        in_specs=[
            pl.BlockSpec((R, F), lambda i: (i, 0)),
            pl.BlockSpec((R, F), lambda i: (i, 0)),
            pl.BlockSpec((R, 1), lambda i: (i, 0)),
        ],
        out_specs=[
            pl.BlockSpec((R, F), lambda i: (i, 0)),
            pl.BlockSpec((R, F), lambda i: (i, 0)),
        ],
        out_shape=[
            jax.ShapeDtypeStruct((N, F), jnp.float32),
            jax.ShapeDtypeStruct((N, F), jnp.float32),
        ],
    )(p0, p1, dinv)


def _tc_layer(q0, q1, dinv, tx0, tx1, Wc, b, R, emit_next):
    N, F = tx0.shape
    H = Wc.shape[2]

    def body(q0_ref, q1_ref, dinv_ref, tx0_ref, tx1_ref, w_ref, b_ref, *outs):
        dv = dinv_ref[...]
        tx0v = tx0_ref[...]
        tx2 = -2.0 * dv * (q0_ref[...] + q1_ref[...]) - tx0v
        out = (jnp.dot(tx0v, w_ref[0], preferred_element_type=jnp.float32)
               + jnp.dot(tx1_ref[...], w_ref[1], preferred_element_type=jnp.float32)
               + jnp.dot(tx2, w_ref[2], preferred_element_type=jnp.float32)
               + b_ref[...])
        outs[0][...] = out
        if emit_next:
            hn = jnp.maximum(out, 0.0)
            outs[1][...] = hn
            outs[2][...] = dv * hn

    grid = N // R
    n_out = 3 if emit_next else 1
    return pl.pallas_call(
        body,
        grid=(grid,),
        in_specs=[
            pl.BlockSpec((R, F), lambda i: (i, 0)),
            pl.BlockSpec((R, F), lambda i: (i, 0)),
            pl.BlockSpec((R, 1), lambda i: (i, 0)),
            pl.BlockSpec((R, F), lambda i: (i, 0)),
            pl.BlockSpec((R, F), lambda i: (i, 0)),
            pl.BlockSpec((3, F, H), lambda i: (0, 0, 0)),
            pl.BlockSpec((1, H), lambda i: (0, 0)),
        ],
        out_specs=[pl.BlockSpec((R, H), lambda i: (i, 0))] * n_out,
        out_shape=[jax.ShapeDtypeStruct((N, H), jnp.float32)] * n_out,
    )(q0, q1, dinv, tx0, tx1, Wc, b)


def _tc_pool(h, batch_f, lin_w, lin_b, C):
    N, H = h.shape
    G = 64
    OUT = lin_w.shape[0]
    grid = N // C

    def body(h_ref, b_ref, w_ref, lb_ref, pooled_ref, out_ref, sums, cnt):
        i = pl.program_id(0)
        gids = lax.broadcasted_iota(jnp.int32, (G, C), 0).astype(jnp.float32)
        oh = jnp.where(gids == b_ref[0], 1.0, 0.0)
        psum = jnp.dot(oh, h_ref[...], preferred_element_type=jnp.float32)
        pcnt = jnp.sum(oh, axis=1, keepdims=True)

        @pl.when(i == 0)
        def _():
            sums[...] = psum
            cnt[...] = pcnt

        @pl.when(i > 0)
        def _():
            sums[...] = sums[...] + psum
            cnt[...] = cnt[...] + pcnt

        @pl.when(i == grid - 1)
        def _():
            pooled = sums[...] / jnp.maximum(cnt[...], 1.0)
            pooled_ref[...] = pooled
            out_ref[...] = lax.dot_general(
                pooled, w_ref[...], (((1,), (1,)), ((), ())),
                preferred_element_type=jnp.float32) + lb_ref[...]

    return pl.pallas_call(
        body,
        grid=(grid,),
        in_specs=[
            pl.BlockSpec((C, H), lambda i: (i, 0)),
            pl.BlockSpec((1, 1, C), lambda i: (i, 0, 0)),
            pl.BlockSpec((OUT, H), lambda i: (0, 0)),
            pl.BlockSpec((1, OUT), lambda i: (0, 0)),
        ],
        out_specs=[
            pl.BlockSpec((G, H), lambda i: (0, 0)),
            pl.BlockSpec((G, OUT), lambda i: (0, 0)),
        ],
        out_shape=[
            jax.ShapeDtypeStruct((G, H), jnp.float32),
            jax.ShapeDtypeStruct((G, OUT), jnp.float32),
        ],
        scratch_shapes=[
            pltpu.VMEM((G, H), jnp.float32),
            pltpu.VMEM((G, 1), jnp.float32),
        ],
    )(h, batch_f, lin_w, lin_b)


def kernel(x, edge_index, batch, W1, b1, W2, b2, W3, b3, lin_w, lin_b):
    N, D = x.shape
    E = edge_index.shape[1]
    H = W1.shape[1]
    K = 125
    CH = E // (NW * K)
    R = 2000
    NP = ((N + 127) // 128 + 1) * 128
    RPT = NP // NS

    rows3 = edge_index[0].reshape(NW, CH, K)
    cols3 = edge_index[1].reshape(NW, CH, K)
    JL = 128 - K
    spread = jnp.arange(NW * CH * JL, dtype=jnp.int32)
    pad3 = (N + spread % (NP - N)).reshape(NW, CH, JL)
    padg = (spread % N).reshape(NW, CH, JL)
    rows3p = jnp.concatenate([rows3, padg], axis=2)
    cols3p = jnp.concatenate([cols3, pad3], axis=2)
    rowsdeg = jnp.concatenate([rows3, pad3], axis=2)
    zeros_h = jnp.zeros((RPT, H), jnp.float32)
    ones_h = jnp.ones((128, 128), jnp.float32)

    deg_k = _make_deg(NP, E, CH, K)
    prop_k = _make_prop(NP, E, CH)

    dpart = deg_k(rowsdeg, ones_h, zeros_h)
    dinv, g = _tc_prep(dpart[0, :N], dpart[1, :N], x, R)

    xs = []
    h = x
    for li, (W, b) in enumerate(((W1, b1), (W2, b2), (W3, b3))):
        Wc = jnp.transpose(W, (0, 2, 1))
        p = prop_k(rows3p, cols3p, g, zeros_h)
        tx1, g1 = _tc_mid(p[0, :N], p[1, :N], dinv, R)
        q = prop_k(rows3p, cols3p, g1, zeros_h)
        last = li == 2
        outs = _tc_layer(q[0, :N], q[1, :N], dinv, h, tx1, Wc,
                         b.reshape(1, H), R, emit_next=not last)
        xs.append(outs[0])
        if not last:
            h, g = outs[1], outs[2]

    h3 = xs[2]
    pooled, out = _tc_pool(h3, batch.astype(jnp.float32).reshape(N // 2000, 1, 2000),
                           lin_w, lin_b.reshape(1, lin_w.shape[0]), 2000)
    return (out, xs[0], xs[1], h3, pooled)


# speedup vs baseline: 2.5278x; 1.0505x over previous
import functools

import jax
import jax.numpy as jnp
from jax import lax
from jax.experimental import pallas as pl
from jax.experimental.pallas import tpu as pltpu
from jax.experimental.pallas import tpu_sc as plsc

NC = 2
NS = 16
NW = NC * NS


@functools.lru_cache(maxsize=None)
def _make_deg(N, E, CH, K):
    RPT = N // NS
    mesh = plsc.VectorSubcoreMesh(core_axis_name="c", subcore_axis_name="s")

    FIRE = 5
    assert CH % FIRE == 0

    @functools.partial(
        pl.kernel,
        out_type=jax.ShapeDtypeStruct((NC, N, 128), jnp.float32),
        mesh=mesh,
        scratch_types=[
            pltpu.VMEM((CH, 128), jnp.int32),
            pltpu.VMEM((128, 128), jnp.float32),
            pltpu.VMEM_SHARED((N, 128), jnp.float32),
            pltpu.SemaphoreType.DMA,
        ],
    )
    def deg_kernel(rows_hbm, ones_hbm, zeros_hbm, out_hbm, ridx, ones_v, acc,
                   ssem):
        c = lax.axis_index("c")
        s = lax.axis_index("s")
        w = c * NS + s
        pltpu.sync_copy(rows_hbm.at[w], ridx)
        pltpu.sync_copy(ones_hbm, ones_v)
        pltpu.sync_copy(zeros_hbm, acc.at[pl.ds(s * RPT, RPT)])
        plsc.subcore_barrier()

        def body(jj, carry):
            for t in range(FIRE):
                pltpu.async_copy(ones_v, acc.at[ridx.at[jj * FIRE + t]], ssem,
                                 add=True)
            for t in range(FIRE):
                pltpu.make_async_copy(ones_v, acc.at[ridx.at[0]], ssem).wait()
            return carry

        lax.fori_loop(0, CH // FIRE, body, 0)
        plsc.subcore_barrier()
        pltpu.sync_copy(acc.at[pl.ds(s * RPT, RPT)],
                        out_hbm.at[c, pl.ds(s * RPT, RPT)])

    return deg_kernel


@functools.lru_cache(maxsize=None)
def _make_prop(N, E, CH):
    RPT = N // NS
    NQ = CH // 4
    assert CH % 8 == 0 and CH >= 16
    mesh = plsc.VectorSubcoreMesh(core_axis_name="c", subcore_axis_name="s")

    @functools.partial(
        pl.kernel,
        out_type=jax.ShapeDtypeStruct((NC, N, 128), jnp.float32),
        mesh=mesh,
        scratch_types=[
            pltpu.VMEM((8, 128), jnp.int32),
            pltpu.VMEM((8, 128), jnp.int32),
            pltpu.VMEM((128, 128), jnp.float32),
            pltpu.VMEM((128, 128), jnp.float32),
            pltpu.SemaphoreType.DMA,
            pltpu.SemaphoreType.DMA,
            pltpu.SemaphoreType.DMA,
            pltpu.SemaphoreType.DMA,
            pltpu.SemaphoreType.DMA,
            pltpu.VMEM_SHARED((N, 128), jnp.float32),
        ],
    )
    def prop_kernel(rows_hbm, cols_hbm, table_hbm, zeros_hbm, out_hbm,
                    rring, cring, buf0, buf1, gs0, gs1, ss0, ss1, isem, acc):
        c = lax.axis_index("c")
        s = lax.axis_index("s")
        w = c * NS + s
        pltpu.sync_copy(zeros_hbm, acc.at[pl.ds(s * RPT, RPT)])
        pltpu.sync_copy(rows_hbm.at[w, pl.ds(0, 4)], rring.at[pl.ds(0, 4)])
        pltpu.sync_copy(cols_hbm.at[w, pl.ds(0, 4)], cring.at[pl.ds(0, 4)])
        plsc.subcore_barrier()

        bufs = (buf0, buf1)
        gsems = (gs0, gs1)
        ssems = (ss0, ss1)

        def gather(r, b):
            pltpu.async_copy(table_hbm.at[rring.at[r]], bufs[b], gsems[b])

        def scatter(r, b):
            pltpu.async_copy(bufs[b], acc.at[cring.at[r]], ssems[b], add=True)

        def wait_g(b):
            pltpu.make_async_copy(table_hbm.at[rring.at[0]], bufs[b],
                                  gsems[b]).wait()

        def wait_s(b):
            pltpu.make_async_copy(bufs[b], acc.at[cring.at[0]],
                                  ssems[b]).wait()

        def slab_issue(start, h):
            pltpu.async_copy(rows_hbm.at[w, pl.ds(start, 4)],
                             rring.at[pl.ds(4 * h, 4)], isem)
            pltpu.async_copy(cols_hbm.at[w, pl.ds(start, 4)],
                             cring.at[pl.ds(4 * h, 4)], isem)

        def slab_wait():
            for _ in range(2):
                pltpu.make_async_copy(rows_hbm.at[w, pl.ds(0, 4)],
                                      rring.at[pl.ds(0, 4)], isem).wait()

        slab_issue(4, 1)
        gather(0, 0)
        wait_g(0)
        gather(1, 1)
        scatter(0, 0)
        wait_g(1); wait_s(0); gather(2, 0); scatter(1, 1)
        wait_g(0); wait_s(1); gather(3, 1); scatter(2, 0)
        wait_g(1); wait_s(0); slab_wait(); gather(4, 0)
        slab_issue(8, 0)
        scatter(3, 1)

        def quad(q, h):
            wait_g(0); wait_s(1); gather(4 * h + 1, 1); scatter(4 * h + 0, 0)
            wait_g(1); wait_s(0); gather(4 * h + 2, 0); scatter(4 * h + 1, 1)
            wait_g(0); wait_s(1); gather(4 * h + 3, 1); scatter(4 * h + 2, 0)
            wait_g(1); wait_s(0); slab_wait()
            gather(4 * (1 - h), 0)
            slab_issue(jnp.minimum(4 * q + 8, CH - 4), h)
            scatter(4 * h + 3, 1)

        def pair(qq, carry):
            quad(2 * qq + 1, 1)
            quad(2 * qq + 2, 0)
            return carry

        lax.fori_loop(0, (NQ - 2) // 2, pair, 0)

        wait_g(0); wait_s(1); gather(5, 1); scatter(4, 0)
        wait_g(1); wait_s(0); gather(6, 0); scatter(5, 1)
        wait_g(0); wait_s(1); gather(7, 1); scatter(6, 0)
        wait_g(1); wait_s(0); slab_wait()
        scatter(7, 1)
        wait_s(1)

        plsc.subcore_barrier()
        pltpu.sync_copy(acc.at[pl.ds(s * RPT, RPT)],
                        out_hbm.at[c, pl.ds(s * RPT, RPT)])

    return prop_kernel


def _tc_prep(dpart, x, R):
    N, F = x.shape

    def body(d0_ref, d1_ref, x_ref, dinv_ref, g0_ref):
        deg = d0_ref[0][:, 0:1] + d1_ref[0][:, 0:1]
        dinv = jnp.where(deg > 0.0, lax.rsqrt(jnp.maximum(deg, 1e-30)), 0.0)
        dinv_ref[...] = dinv
        g0_ref[...] = dinv * x_ref[...]

    grid = N // R
    return pl.pallas_call(
        body,
        grid=(grid,),
        in_specs=[
            pl.BlockSpec((1, R, 128), lambda i: (0, i, 0)),
            pl.BlockSpec((1, R, 128), lambda i: (1, i, 0)),
            pl.BlockSpec((R, F), lambda i: (i, 0)),
        ],
        out_specs=[
            pl.BlockSpec((R, 1), lambda i: (i, 0)),
            pl.BlockSpec((R, F), lambda i: (i, 0)),
        ],
        out_shape=[
            jax.ShapeDtypeStruct((N, 1), jnp.float32),
            jax.ShapeDtypeStruct((N, F), jnp.float32),
        ],
    )(dpart, dpart, x)


def _tc_mid(p, dinv, R):
    _, NP_, F = p.shape
    N = dinv.shape[0]

    def body(p0_ref, p1_ref, dinv_ref, tx1_ref, g1_ref):
        dv = dinv_ref[...]
        tx1 = -dv * (p0_ref[0] + p1_ref[0])
        tx1_ref[...] = tx1
        g1_ref[...] = dv * tx1

    grid = N // R
    return pl.pallas_call(
        body,
        grid=(grid,),
        in_specs=[
            pl.BlockSpec((1, R, F), lambda i: (0, i, 0)),
            pl.BlockSpec((1, R, F), lambda i: (1, i, 0)),
            pl.BlockSpec((R, 1), lambda i: (i, 0)),
        ],
        out_specs=[
            pl.BlockSpec((R, F), lambda i: (i, 0)),
            pl.BlockSpec((R, F), lambda i: (i, 0)),
        ],
        out_shape=[
            jax.ShapeDtypeStruct((N, F), jnp.float32),
            jax.ShapeDtypeStruct((N, F), jnp.float32),
        ],
    )(p, p, dinv)


def _tc_layer(q, dinv, tx0, tx1, Wc, b, R, emit_next):
    N, F = tx0.shape
    H = Wc.shape[2]

    def body(q0_ref, q1_ref, dinv_ref, tx0_ref, tx1_ref, w_ref, b_ref, *outs):
        dv = dinv_ref[...]
        tx0v = tx0_ref[...]
        tx2 = -2.0 * dv * (q0_ref[0] + q1_ref[0]) - tx0v
        out = (jnp.dot(tx0v, w_ref[0], preferred_element_type=jnp.float32)
               + jnp.dot(tx1_ref[...], w_ref[1], preferred_element_type=jnp.float32)
               + jnp.dot(tx2, w_ref[2], preferred_element_type=jnp.float32)
               + b_ref[...])
        outs[0][...] = out
        if emit_next:
            hn = jnp.maximum(out, 0.0)
            outs[1][...] = hn
            outs[2][...] = dv * hn

    grid = N // R
    n_out = 3 if emit_next else 1
    return pl.pallas_call(
        body,
        grid=(grid,),
        in_specs=[
            pl.BlockSpec((1, R, F), lambda i: (0, i, 0)),
            pl.BlockSpec((1, R, F), lambda i: (1, i, 0)),
            pl.BlockSpec((R, 1), lambda i: (i, 0)),
            pl.BlockSpec((R, F), lambda i: (i, 0)),
            pl.BlockSpec((R, F), lambda i: (i, 0)),
            pl.BlockSpec((3, F, H), lambda i: (0, 0, 0)),
            pl.BlockSpec((1, H), lambda i: (0, 0)),
        ],
        out_specs=[pl.BlockSpec((R, H), lambda i: (i, 0))] * n_out,
        out_shape=[jax.ShapeDtypeStruct((N, H), jnp.float32)] * n_out,
    )(q, q, dinv, tx0, tx1, Wc, b)


def _tc_pool(h, batch_f, lin_w, lin_b, C):
    N, H = h.shape
    G = 64
    OUT = lin_w.shape[0]
    grid = N // C

    def body(h_ref, b_ref, w_ref, lb_ref, pooled_ref, out_ref, sums, cnt):
        i = pl.program_id(0)
        gids = lax.broadcasted_iota(jnp.int32, (G, C), 0).astype(jnp.float32)
        oh = jnp.where(gids == b_ref[0], 1.0, 0.0)
        psum = jnp.dot(oh, h_ref[...], preferred_element_type=jnp.float32)
        pcnt = jnp.sum(oh, axis=1, keepdims=True)

        @pl.when(i == 0)
        def _():
            sums[...] = psum
            cnt[...] = pcnt

        @pl.when(i > 0)
        def _():
            sums[...] = sums[...] + psum
            cnt[...] = cnt[...] + pcnt

        @pl.when(i == grid - 1)
        def _():
            pooled = sums[...] / jnp.maximum(cnt[...], 1.0)
            pooled_ref[...] = pooled
            out_ref[...] = lax.dot_general(
                pooled, w_ref[...], (((1,), (1,)), ((), ())),
                preferred_element_type=jnp.float32) + lb_ref[...]

    return pl.pallas_call(
        body,
        grid=(grid,),
        in_specs=[
            pl.BlockSpec((C, H), lambda i: (i, 0)),
            pl.BlockSpec((1, 1, C), lambda i: (i, 0, 0)),
            pl.BlockSpec((OUT, H), lambda i: (0, 0)),
            pl.BlockSpec((1, OUT), lambda i: (0, 0)),
        ],
        out_specs=[
            pl.BlockSpec((G, H), lambda i: (0, 0)),
            pl.BlockSpec((G, OUT), lambda i: (0, 0)),
        ],
        out_shape=[
            jax.ShapeDtypeStruct((G, H), jnp.float32),
            jax.ShapeDtypeStruct((G, OUT), jnp.float32),
        ],
        scratch_shapes=[
            pltpu.VMEM((G, H), jnp.float32),
            pltpu.VMEM((G, 1), jnp.float32),
        ],
    )(h, batch_f, lin_w, lin_b)


def kernel(x, edge_index, batch, W1, b1, W2, b2, W3, b3, lin_w, lin_b):
    N, D = x.shape
    E = edge_index.shape[1]
    H = W1.shape[1]
    K = 125
    CH = E // (NW * K)
    R = 2000
    NP = ((N + 127) // 128 + 1) * 128
    RPT = NP // NS

    rows3 = edge_index[0].reshape(NW, CH, K)
    cols3 = edge_index[1].reshape(NW, CH, K)
    JL = 128 - K
    spread = jnp.arange(NW * CH * JL, dtype=jnp.int32)
    pad3 = (N + spread % (NP - N)).reshape(NW, CH, JL)
    padg = (spread % N).reshape(NW, CH, JL)
    rows3p = jnp.concatenate([rows3, padg], axis=2)
    cols3p = jnp.concatenate([cols3, pad3], axis=2)
    rowsdeg = jnp.concatenate([rows3, pad3], axis=2)
    zeros_h = jnp.zeros((RPT, H), jnp.float32)
    ones_h = jnp.ones((128, 128), jnp.float32)

    deg_k = _make_deg(NP, E, CH, K)
    prop_k = _make_prop(NP, E, CH)

    dpart = deg_k(rowsdeg, ones_h, zeros_h)
    dinv, g = _tc_prep(dpart, x, R)

    xs = []
    h = x
    for li, (W, b) in enumerate(((W1, b1), (W2, b2), (W3, b3))):
        Wc = jnp.transpose(W, (0, 2, 1))
        p = prop_k(rows3p, cols3p, g, zeros_h)
        tx1, g1 = _tc_mid(p, dinv, R)
        q = prop_k(rows3p, cols3p, g1, zeros_h)
        last = li == 2
        outs = _tc_layer(q, dinv, h, tx1, Wc,
                         b.reshape(1, H), R, emit_next=not last)
        xs.append(outs[0])
        if not last:
            h, g = outs[1], outs[2]

    h3 = xs[2]
    pooled, out = _tc_pool(h3, batch.astype(jnp.float32).reshape(N // 2000, 1, 2000),
                           lin_w, lin_b.reshape(1, lin_w.shape[0]), 2000)
    return (out, xs[0], xs[1], h3, pooled)
